# baseline (speedup 1.0000x reference)
import functools

import jax
import jax.numpy as jnp
from jax.experimental import pallas as pl
from jax.experimental.pallas import tpu as pltpu

_BN_EPS = 1e-5


def _fused_kernel(x_ref, w1_ref, bgb_ref, w2_ref, o_ref,
                  stats_ref, w2s_ref, c_ref, h_ref, *, batch_size,
                  d_out, tb):
    phase = pl.program_id(0)
    i = pl.program_id(1)

    @pl.when(phase == 0)
    def _gemm1_and_stats():
        nch = max(tb // 1024, 1)
        half = tb // nch
        start = pl.multiple_of(i * tb, tb)
        s8 = None
        q8 = None
        for k in range(nch):
            hk = jnp.dot(x_ref[pl.ds(k * half, half), :], w1_ref[...],
                         preferred_element_type=jnp.float32)
            hk = jnp.maximum(hk + bgb_ref[0:1, :], 0.0)
            h_ref[pl.ds(start + k * half, half), :] = hk.astype(h_ref.dtype)
            hr = hk.reshape(half // 8, 8, hk.shape[1])
            sk = jnp.sum(hr, axis=0)
            qk = jnp.sum(hr * hr, axis=0)
            s8 = sk if s8 is None else s8 + sk
            q8 = qk if q8 is None else q8 + qk

        @pl.when(i == 0)
        def _init():
            stats_ref[0:8, :] = s8
            stats_ref[8:16, :] = q8

        @pl.when(i > 0)
        def _acc():
            stats_ref[0:8, :] += s8
            stats_ref[8:16, :] += q8

    @pl.when(phase == 1)
    def _gemm2():
        @pl.when(i == 0)
        def _finalize():
            inv_b = 1.0 / batch_size
            mean = jnp.sum(stats_ref[0:8, :], axis=0, keepdims=True) * inv_b
            msq = jnp.sum(stats_ref[8:16, :], axis=0, keepdims=True) * inv_b
            var = jnp.maximum(msq - mean * mean, 0.0)
            scale = jax.lax.rsqrt(var + _BN_EPS) * bgb_ref[1:2, :]
            shift = bgb_ref[2:3, :] - mean * scale
            scale_c = scale.reshape(scale.shape[1], 1)
            shift_c = shift.reshape(shift.shape[1], 1)
            w2s_ref[...] = (w2_ref[...] * scale_c).astype(w2s_ref.dtype)
            c_ref[...] = (jnp.sum(w2_ref[...] * shift_c, axis=0,
                                  keepdims=True) + bgb_ref[3:4, 0:d_out])

        start = pl.multiple_of(i * tb, tb)
        h = h_ref[pl.ds(start, tb), :]
        out = jnp.dot(h, w2s_ref[...], preferred_element_type=jnp.float32)
        o_ref[...] = jnp.maximum(out + c_ref[...], 0.0).astype(o_ref.dtype)


def _pick_tile(batch, block_b):
    if batch <= block_b:
        return batch
    if batch % block_b == 0:
        return block_b
    for t in range(block_b, 7, -1):
        if batch % t == 0 and t % 8 == 0:
            return t
    return batch


def kernel(x, w1, b1, gamma, beta, w2, b2):
    B, d_in = x.shape
    H = w1.shape[1]
    d_out = w2.shape[1]

    bgb = jnp.concatenate(
        [b1.reshape(1, H), gamma.reshape(1, H), beta.reshape(1, H),
         jnp.pad(b2.reshape(1, d_out), ((0, 0), (0, H - d_out)))], axis=0)

    tb = _pick_tile(B, 4096)
    nb = B // tb

    x_map = lambda p, i: ((1 - p) * i + p * (nb - 1), 0)

    return pl.pallas_call(
        functools.partial(_fused_kernel, batch_size=B, d_out=d_out, tb=tb),
        out_shape=jax.ShapeDtypeStruct((B, d_out), jnp.float32),
        grid=(2, nb),
        in_specs=[
            pl.BlockSpec((tb, d_in), x_map),
            pl.BlockSpec((d_in, H), lambda p, i: (0, 0)),
            pl.BlockSpec((4, H), lambda p, i: (0, 0)),
            pl.BlockSpec((H, d_out), lambda p, i: (0, 0)),
        ],
        out_specs=pl.BlockSpec((tb, d_out), lambda p, i: (p * i, 0)),
        scratch_shapes=[
            pltpu.VMEM((16, H), jnp.float32),
            pltpu.VMEM((H, d_out), jnp.bfloat16),
            pltpu.VMEM((1, d_out), jnp.float32),
            pltpu.VMEM((B, H), jnp.bfloat16),
        ],
        compiler_params=pltpu.CompilerParams(
            dimension_semantics=("arbitrary", "arbitrary"),
            allow_input_fusion=[False, False, True, False],
            vmem_limit_bytes=48 * 1024 * 1024,
        ),
    )(x, w1, bgb, w2)


# speedup vs baseline: 1.0269x; 1.0269x over previous
import functools

import jax
import jax.numpy as jnp
from jax.experimental import pallas as pl
from jax.experimental.pallas import tpu as pltpu

_BN_EPS = 1e-5


def _fused_kernel(x_ref, w1_ref, bgb_ref, w2_ref, o_ref,
                  stats_ref, w2s_ref, c_ref, h_ref, *, batch_size,
                  d_out, tb):
    phase = pl.program_id(0)
    i = pl.program_id(1)

    @pl.when(phase == 0)
    def _gemm1_and_stats():
        nch = max(tb // 1024, 1)
        half = tb // nch
        start = pl.multiple_of(i * tb, tb)
        s8 = None
        q8 = None
        for k in range(nch):
            hk = jnp.dot(x_ref[pl.ds(k * half, half), :], w1_ref[...],
                         preferred_element_type=jnp.float32)
            hk = jnp.maximum(hk + bgb_ref[0:1, :], 0.0)
            h_ref[pl.ds(start + k * half, half), :] = hk.astype(h_ref.dtype)
            hr = hk.reshape(half // 8, 8, hk.shape[1])
            sk = jnp.sum(hr, axis=0)
            qk = jnp.sum(hr * hr, axis=0)
            s8 = sk if s8 is None else s8 + sk
            q8 = qk if q8 is None else q8 + qk

        @pl.when(i == 0)
        def _init():
            stats_ref[0:8, :] = s8
            stats_ref[8:16, :] = q8

        @pl.when(i > 0)
        def _acc():
            stats_ref[0:8, :] += s8
            stats_ref[8:16, :] += q8

    @pl.when(phase == 1)
    def _gemm2():
        @pl.when(i == 0)
        def _finalize():
            inv_b = 1.0 / batch_size
            mean = jnp.sum(stats_ref[0:8, :], axis=0, keepdims=True) * inv_b
            msq = jnp.sum(stats_ref[8:16, :], axis=0, keepdims=True) * inv_b
            var = jnp.maximum(msq - mean * mean, 0.0)
            scale = jax.lax.rsqrt(var + _BN_EPS) * bgb_ref[1:2, :]
            shift = bgb_ref[2:3, :] - mean * scale
            scale_c = scale.reshape(scale.shape[1], 1)
            shift_c = shift.reshape(shift.shape[1], 1)
            w2s_ref[...] = (w2_ref[...] * scale_c).astype(w2s_ref.dtype)
            c_ref[...] = (jnp.sum(w2_ref[...] * shift_c, axis=0,
                                  keepdims=True) + bgb_ref[3:4, 0:d_out])

        start = pl.multiple_of(i * tb, tb)
        h = h_ref[pl.ds(start, tb), :]
        out = jnp.dot(h, w2s_ref[...], preferred_element_type=jnp.float32)
        o_ref[...] = jnp.maximum(out + c_ref[...], 0.0).astype(o_ref.dtype)


def _pick_tile(batch, block_b):
    if batch <= block_b:
        return batch
    if batch % block_b == 0:
        return block_b
    for t in range(block_b, 7, -1):
        if batch % t == 0 and t % 8 == 0:
            return t
    return batch


def kernel(x, w1, b1, gamma, beta, w2, b2):
    B, d_in = x.shape
    H = w1.shape[1]
    d_out = w2.shape[1]

    bgb = jnp.concatenate(
        [b1.reshape(1, H), gamma.reshape(1, H), beta.reshape(1, H),
         jnp.pad(b2.reshape(1, d_out), ((0, 0), (0, H - d_out)))], axis=0)

    tb = _pick_tile(B, 4096)
    nb = B // tb

    x_map = lambda p, i: ((1 - p) * i + p * (nb - 1), 0)

    return pl.pallas_call(
        functools.partial(_fused_kernel, batch_size=B, d_out=d_out, tb=tb),
        out_shape=jax.ShapeDtypeStruct((B, d_out), jnp.float32),
        grid=(2, nb),
        in_specs=[
            pl.BlockSpec((tb, d_in), x_map),
            pl.BlockSpec((d_in, H), lambda p, i: (0, 0)),
            pl.BlockSpec((4, H), lambda p, i: (0, 0)),
            pl.BlockSpec((H, d_out), lambda p, i: (0, 0)),
        ],
        out_specs=pl.BlockSpec((tb, d_out), lambda p, i: (p * i, 0)),
        scratch_shapes=[
            pltpu.VMEM((16, H), jnp.float32),
            pltpu.VMEM((H, d_out), jnp.float32),
            pltpu.VMEM((1, d_out), jnp.float32),
            pltpu.VMEM((B, H), jnp.float32),
        ],
        compiler_params=pltpu.CompilerParams(
            dimension_semantics=("arbitrary", "arbitrary"),
            allow_input_fusion=[False, False, True, False],
            vmem_limit_bytes=48 * 1024 * 1024,
        ),
    )(x, w1, bgb, w2)
